# baseline (speedup 1.0000x reference)
import jax
import jax.numpy as jnp
from jax.experimental import pallas as pl
from jax.experimental.pallas import tpu as pltpu

_BLOCK_ROWS = 8000


def _body(x_ref, wt_ref, b_ref, o_ref):
    o_ref[...] = (
        jnp.dot(x_ref[...], wt_ref[...], preferred_element_type=jnp.float32)
        + b_ref[...]
    )


def kernel(x, W, b):
    n, in_dim = x.shape
    emb_dim = W.shape[0]
    wt = W.T
    b2 = b.reshape(1, emb_dim)
    grid = n // _BLOCK_ROWS
    return pl.pallas_call(
        _body,
        grid=(grid,),
        in_specs=[
            pl.BlockSpec((_BLOCK_ROWS, in_dim), lambda i: (i, 0)),
            pl.BlockSpec((in_dim, emb_dim), lambda i: (0, 0)),
            pl.BlockSpec((1, emb_dim), lambda i: (0, 0)),
        ],
        out_specs=pl.BlockSpec((_BLOCK_ROWS, emb_dim), lambda i: (i, 0)),
        out_shape=jax.ShapeDtypeStruct((n, emb_dim), jnp.float32),
        compiler_params=pltpu.CompilerParams(
            dimension_semantics=("parallel",),
        ),
    )(x, wt, b2)
